# baseline (speedup 1.0000x reference)
import jax
import jax.numpy as jnp
from jax import lax
from jax.experimental import pallas as pl
from jax.experimental.pallas import tpu as pltpu
from jax.experimental.pallas import tpu_sc as plsc

_VOCAB = 100000
_EMBED = 64
_BATCH = 4096
_SEQ = 200


_CHUNK = 4096
_NFULL = 24
_MAIN_W = 1664
_TAIL_W = 32
_LAST_W = 1792
_PROJ_PAD = _NFULL * _CHUNK + _LAST_W
_NBUF = 8


def _proj_body(embT_hbm, w_ref, tail_ref, b_ref, out_hbm, *scratch):
    bufs = scratch[:_NBUF]
    obufs = scratch[_NBUF : 2 * _NBUF]
    isems = scratch[2 * _NBUF : 3 * _NBUF]
    osems = scratch[3 * _NBUF :]

    def in_copy(c, s):
        if c < _NFULL:
            return pltpu.make_async_copy(
                embT_hbm.at[:, pl.ds(c * _CHUNK, _CHUNK)], bufs[s], isems[s]
            )
        return pltpu.make_async_copy(
            embT_hbm.at[:, pl.ds(_NFULL * _CHUNK, _MAIN_W)],
            bufs[s].at[:, pl.ds(0, _MAIN_W)],
            isems[s],
        )

    for c in range(_NBUF):
        in_copy(c, c).start()
    for c in range(_NFULL + 1):
        s = c % _NBUF
        in_copy(c, s).wait()
        width = _CHUNK if c < _NFULL else _MAIN_W
        p = lax.dot_general(
            w_ref[...],
            bufs[s][:, :width],
            dimension_numbers=(((1,), (0,)), ((), ())),
            preferred_element_type=jnp.float32,
            precision=lax.Precision.DEFAULT,
        )
        if c == 0:
            lane = lax.broadcasted_iota(jnp.int32, (1, _CHUNK), 1)
            p = jnp.where(lane == 0, 0.0, p)
        nxt = c + _NBUF
        if nxt < _NFULL + 1:
            in_copy(nxt, s).start()
        if c >= _NBUF:
            pltpu.make_async_copy(
                obufs[s].at[0],
                out_hbm.at[pl.ds((c - _NBUF) * _CHUNK, _CHUNK)],
                osems[s],
            ).wait()
        if c < _NFULL:
            obufs[s][...] = p + b_ref[0, 0]
            pltpu.async_copy(
                obufs[s].at[0], out_hbm.at[pl.ds(c * _CHUNK, _CHUNK)], osems[s]
            )
        else:
            obufs[s][0, pl.ds(0, _MAIN_W)] = (p + b_ref[0, 0])[0]
            p_tail = lax.dot_general(
                w_ref[...],
                tail_ref[...],
                dimension_numbers=(((1,), (1,)), ((), ())),
                preferred_element_type=jnp.float32,
                precision=lax.Precision.DEFAULT,
            )
            obufs[s][0, pl.ds(_MAIN_W, _TAIL_W)] = (p_tail + b_ref[0, 0])[0]
            pltpu.async_copy(
                obufs[s].at[0, pl.ds(0, _LAST_W)],
                out_hbm.at[pl.ds(_NFULL * _CHUNK, _LAST_W)],
                osems[s],
            )
    for c in range(_NFULL + 1 - _NBUF, _NFULL + 1):
        s = c % _NBUF
        if c < _NFULL:
            pltpu.make_async_copy(
                obufs[s].at[0], out_hbm.at[pl.ds(c * _CHUNK, _CHUNK)], osems[s]
            ).wait()
        else:
            pltpu.make_async_copy(
                obufs[s].at[0, pl.ds(0, _LAST_W)],
                out_hbm.at[pl.ds(_NFULL * _CHUNK, _LAST_W)],
                osems[s],
            ).wait()


def _project(emb, fc_w, fc_b):
    embT = emb.T
    w2 = fc_w.reshape(1, _EMBED)
    tail = lax.slice(emb, (_VOCAB - _TAIL_W, 0), (_VOCAB, _EMBED))
    b2 = fc_b.reshape(1, 1)
    return pl.pallas_call(
        _proj_body,
        in_specs=[
            pl.BlockSpec(memory_space=pl.ANY),
            pl.BlockSpec(memory_space=pltpu.MemorySpace.VMEM),
            pl.BlockSpec(memory_space=pltpu.MemorySpace.VMEM),
            pl.BlockSpec(memory_space=pltpu.MemorySpace.VMEM),
        ],
        out_specs=pl.BlockSpec(memory_space=pl.ANY),
        out_shape=jax.ShapeDtypeStruct((_PROJ_PAD,), jnp.float32),
        scratch_shapes=(
            [pltpu.VMEM((_EMBED, _CHUNK), jnp.float32) for _ in range(_NBUF)]
            + [pltpu.VMEM((1, _CHUNK), jnp.float32) for _ in range(_NBUF)]
            + [pltpu.SemaphoreType.DMA for _ in range(2 * _NBUF)]
        ),
    )(embT, w2, tail, b2)


_NC = 2
_NS = 16
_NW = _NC * _NS
_RPT = _BATCH // _NW
_L = 16
_G = _RPT // _L


def _sc_body(proj_hbm, x_hbm, out_hbm, proj_v, x_v, out_v, sem_p, sem_x):
    wid = lax.axis_index("s") * _NC + lax.axis_index("c")
    base = wid * _RPT
    cp = pltpu.async_copy(proj_hbm, proj_v, sem_p)
    cx = pltpu.async_copy(x_hbm.at[pl.ds(base * _SEQ, _RPT * _SEQ)], x_v, sem_x)
    cp.wait()
    cx.wait()

    lanes = lax.iota(jnp.int32, _L)
    rows = tuple((g * _L + lanes) * _SEQ for g in range(_G))

    def body(t, accs):
        new = []
        for g in range(_G):
            idx = plsc.load_gather(x_v, [rows[g] + t])
            vals = plsc.load_gather(proj_v, [idx])
            new.append(accs[g] + vals)
        return tuple(new)

    accs0 = tuple(jnp.zeros((_L,), jnp.float32) for _ in range(_G))
    accs = lax.fori_loop(0, _SEQ, body, accs0, unroll=2)

    for g in range(_G):
        z = accs[g] * (1.0 / _SEQ)
        out_v[pl.ds(g * _L, _L)] = 1.0 / (1.0 + jnp.exp(-z))
    pltpu.sync_copy(out_v, out_hbm.at[pl.ds(base, _RPT)])


_sc_call = pl.kernel(
    _sc_body,
    out_type=jax.ShapeDtypeStruct((_BATCH,), jnp.float32),
    mesh=plsc.VectorSubcoreMesh(core_axis_name="c", subcore_axis_name="s"),
    compiler_params=pltpu.CompilerParams(needs_layout_passes=False),
    scratch_types=[
        pltpu.VMEM((_PROJ_PAD,), jnp.float32),
        pltpu.VMEM((_RPT * _SEQ,), jnp.int32),
        pltpu.VMEM((_RPT,), jnp.float32),
        pltpu.SemaphoreType.DMA,
        pltpu.SemaphoreType.DMA,
    ],
)


def kernel(x, emb, fc_w, fc_b):
    proj = _project(emb, fc_w, fc_b)
    return _sc_call(proj, x.astype(jnp.int32).reshape(_BATCH * _SEQ))


# speedup vs baseline: 1.3104x; 1.3104x over previous
import jax
import jax.numpy as jnp
from jax import lax
from jax.experimental import pallas as pl
from jax.experimental.pallas import tpu as pltpu
from jax.experimental.pallas import tpu_sc as plsc

_VOCAB = 100000
_EMBED = 64
_BATCH = 4096
_SEQ = 200


_CHUNK = 4096
_NFULL = 24
_MAIN_W = 1664
_TAIL_W = 32
_LAST_W = 1792
_PROJ_PAD = _NFULL * _CHUNK + _LAST_W
_NBUF = 8


def _proj_body(embT_hbm, w_ref, tail_ref, b_ref, out_hbm, *scratch):
    bufs = scratch[:_NBUF]
    obufs = scratch[_NBUF : 2 * _NBUF]
    isems = scratch[2 * _NBUF : 3 * _NBUF]
    osems = scratch[3 * _NBUF :]

    def in_copy(c, s):
        if c < _NFULL:
            return pltpu.make_async_copy(
                embT_hbm.at[:, pl.ds(c * _CHUNK, _CHUNK)], bufs[s], isems[s]
            )
        return pltpu.make_async_copy(
            embT_hbm.at[:, pl.ds(_NFULL * _CHUNK, _MAIN_W)],
            bufs[s].at[:, pl.ds(0, _MAIN_W)],
            isems[s],
        )

    for c in range(_NBUF):
        in_copy(c, c).start()
    for c in range(_NFULL + 1):
        s = c % _NBUF
        in_copy(c, s).wait()
        width = _CHUNK if c < _NFULL else _MAIN_W
        p = lax.dot_general(
            w_ref[...],
            bufs[s][:, :width],
            dimension_numbers=(((1,), (0,)), ((), ())),
            preferred_element_type=jnp.float32,
            precision=lax.Precision.DEFAULT,
        )
        if c == 0:
            lane = lax.broadcasted_iota(jnp.int32, (1, _CHUNK), 1)
            p = jnp.where(lane == 0, 0.0, p)
        nxt = c + _NBUF
        if nxt < _NFULL + 1:
            in_copy(nxt, s).start()
        if c >= _NBUF:
            pltpu.make_async_copy(
                obufs[s].at[0],
                out_hbm.at[pl.ds((c - _NBUF) * _CHUNK, _CHUNK)],
                osems[s],
            ).wait()
        if c < _NFULL:
            obufs[s][...] = p + b_ref[0, 0]
            pltpu.async_copy(
                obufs[s].at[0], out_hbm.at[pl.ds(c * _CHUNK, _CHUNK)], osems[s]
            )
        else:
            obufs[s][0, pl.ds(0, _MAIN_W)] = (p + b_ref[0, 0])[0]
            p_tail = lax.dot_general(
                w_ref[...],
                tail_ref[...],
                dimension_numbers=(((1,), (1,)), ((), ())),
                preferred_element_type=jnp.float32,
                precision=lax.Precision.DEFAULT,
            )
            obufs[s][0, pl.ds(_MAIN_W, _TAIL_W)] = (p_tail + b_ref[0, 0])[0]
            pltpu.async_copy(
                obufs[s].at[0, pl.ds(0, _LAST_W)],
                out_hbm.at[pl.ds(_NFULL * _CHUNK, _LAST_W)],
                osems[s],
            )
    for c in range(_NFULL + 1 - _NBUF, _NFULL + 1):
        s = c % _NBUF
        if c < _NFULL:
            pltpu.make_async_copy(
                obufs[s].at[0], out_hbm.at[pl.ds(c * _CHUNK, _CHUNK)], osems[s]
            ).wait()
        else:
            pltpu.make_async_copy(
                obufs[s].at[0, pl.ds(0, _LAST_W)],
                out_hbm.at[pl.ds(_NFULL * _CHUNK, _LAST_W)],
                osems[s],
            ).wait()


def _project(emb, fc_w, fc_b):
    embT = emb.T
    w2 = fc_w.reshape(1, _EMBED)
    tail = lax.slice(emb, (_VOCAB - _TAIL_W, 0), (_VOCAB, _EMBED))
    b2 = fc_b.reshape(1, 1)
    return pl.pallas_call(
        _proj_body,
        in_specs=[
            pl.BlockSpec(memory_space=pl.ANY),
            pl.BlockSpec(memory_space=pltpu.MemorySpace.VMEM),
            pl.BlockSpec(memory_space=pltpu.MemorySpace.VMEM),
            pl.BlockSpec(memory_space=pltpu.MemorySpace.VMEM),
        ],
        out_specs=pl.BlockSpec(memory_space=pl.ANY),
        out_shape=jax.ShapeDtypeStruct((_PROJ_PAD,), jnp.float32),
        scratch_shapes=(
            [pltpu.VMEM((_EMBED, _CHUNK), jnp.float32) for _ in range(_NBUF)]
            + [pltpu.VMEM((1, _CHUNK), jnp.float32) for _ in range(_NBUF)]
            + [pltpu.SemaphoreType.DMA for _ in range(2 * _NBUF)]
        ),
    )(embT, w2, tail, b2)


_NC = 2
_NS = 16
_NW = _NC * _NS
_RPT = _BATCH // _NW
_L = 16
_G = _RPT // _L


def _sc_body(xt_hbm, proj_hbm, out_hbm, proj_v, x_v, out_v, sem_p, sem_x):
    wid = lax.axis_index("s") * _NC + lax.axis_index("c")
    base = wid * _RPT
    cp = pltpu.async_copy(proj_hbm, proj_v, sem_p)
    cx = pltpu.async_copy(xt_hbm.at[:, pl.ds(base, _RPT)], x_v, sem_x)
    cp.wait()
    cx.wait()

    def body(t, accs):
        new = []
        for g in range(_G):
            idx = x_v[t, pl.ds(g * _L, _L)]
            vals = plsc.load_gather(proj_v, [idx])
            new.append(accs[g] + vals)
        return tuple(new)

    accs0 = tuple(jnp.zeros((_L,), jnp.float32) for _ in range(_G))
    accs = lax.fori_loop(0, _SEQ, body, accs0, unroll=2)

    for g in range(_G):
        z = accs[g] * (1.0 / _SEQ)
        out_v[pl.ds(g * _L, _L)] = 1.0 / (1.0 + jnp.exp(-z))
    pltpu.sync_copy(out_v, out_hbm.at[pl.ds(base, _RPT)])


_sc_call = pl.kernel(
    _sc_body,
    out_type=jax.ShapeDtypeStruct((_BATCH,), jnp.float32),
    mesh=plsc.VectorSubcoreMesh(core_axis_name="c", subcore_axis_name="s"),
    compiler_params=pltpu.CompilerParams(needs_layout_passes=False),
    scratch_types=[
        pltpu.VMEM((_PROJ_PAD,), jnp.float32),
        pltpu.VMEM((_SEQ, _RPT), jnp.int32),
        pltpu.VMEM((_RPT,), jnp.float32),
        pltpu.SemaphoreType.DMA,
        pltpu.SemaphoreType.DMA,
    ],
)


def kernel(x, emb, fc_w, fc_b):
    proj = _project(emb, fc_w, fc_b)
    return _sc_call(x.astype(jnp.int32).T, proj)
